# baseline (speedup 1.0000x reference)
import functools
import math

import jax
import jax.numpy as jnp
from jax import lax
from jax.experimental import pallas as pl
from jax.experimental.pallas import tpu as pltpu
from jax.experimental.pallas import tpu_sc as plsc


_BT = 1024
_NC, _NS = 2, 16
_NW = _NC * _NS
_CH = 512


def _tc_body(x_ref, w1_ref, b1_ref, g1_ref, be1_ref, w2_ref, b2_ref, g2_ref,
             be2_ref, cb_ref, idx_ref, loss_ref):
    x = x_ref[...]
    h = jnp.dot(x, w1_ref[...], preferred_element_type=jnp.float32)
    h = h + b1_ref[...]
    mu = jnp.mean(h, axis=-1, keepdims=True)
    var = jnp.mean((h - mu) ** 2, axis=-1, keepdims=True)
    h = (h - mu) * jax.lax.rsqrt(var + 1e-5) * g1_ref[...] + be1_ref[...]
    h = 0.5 * h * (1.0 + jax.lax.erf(h * (1.0 / math.sqrt(2.0))))
    z = jnp.dot(h, w2_ref[...], preferred_element_type=jnp.float32)
    z = z + b2_ref[...]
    mu2 = jnp.mean(z, axis=-1, keepdims=True)
    var2 = jnp.mean((z - mu2) ** 2, axis=-1, keepdims=True)
    z = (z - mu2) * jax.lax.rsqrt(var2 + 1e-5) * g2_ref[...] + be2_ref[...]

    cb = cb_ref[...]
    cb2 = jnp.sum(cb * cb, axis=1)
    z2 = jnp.sum(z * z, axis=1, keepdims=True)
    dist = (z2 + cb2[None, :]) - jnp.dot(z + z, cb.T, preferred_element_type=jnp.float32)
    k = cb.shape[0]
    iota = jax.lax.broadcasted_iota(jnp.int32, (1, k), 1).astype(jnp.float32)
    minv = jnp.min(dist, axis=1, keepdims=True)
    idx = jnp.min(jnp.where(dist == minv, iota, float(k)), axis=1).astype(jnp.int32)
    idx_ref[0, 0, :] = idx

    part = jnp.sum(minv).reshape(1, 1)

    @pl.when(pl.program_id(0) == 0)
    def _init():
        loss_ref[...] = jnp.zeros_like(loss_ref)

    loss_ref[...] += part


def _sc_gather(cb_hbm, idx_hbm, zq_hbm, idx0, idx1, rows0, rows1, sem0, sem1):
    n = zq_hbm.shape[0]
    bpw = n // _NW
    nch = bpw // _CH
    wid = lax.axis_index("s") * _NC + lax.axis_index("c")
    base = wid * bpw
    idx_v = (idx0, idx1)
    rows_v = (rows0, rows1)
    sems = (sem0, sem1)

    pltpu.sync_copy(idx_hbm.at[pl.ds(base, _CH)], idx0)
    gathers = [pltpu.async_copy(cb_hbm.at[idx0], rows0, sem0)]
    for c in range(nch):
        cur = c % 2
        nxt = (c + 1) % 2
        if c + 1 < nch:
            off = base + (c + 1) * _CH
            pltpu.sync_copy(idx_hbm.at[pl.ds(off, _CH)], idx_v[nxt])
            gathers.append(
                pltpu.async_copy(cb_hbm.at[idx_v[nxt]], rows_v[nxt], sems[nxt]))
        gathers[c].wait()
        pltpu.sync_copy(rows_v[cur], zq_hbm.at[pl.ds(base + c * _CH, _CH)])


_SPLIT_BLOCKS = (48, 16)


def kernel(embeddings, W1, b1, g1, be1, W2, b2, g2, be2, codebook):
    b, t, in_dim = embeddings.shape
    n = b * t
    lat = codebook.shape[1]
    x = embeddings.reshape(n, in_dim)

    def make_sc_gather(nh):
        return pl.kernel(
            _sc_gather,
            mesh=plsc.VectorSubcoreMesh(core_axis_name="c", subcore_axis_name="s"),
            compiler_params=pltpu.CompilerParams(use_tc_tiling_on_sc=False),
            out_type=jax.ShapeDtypeStruct((nh, lat), jnp.float32),
            scratch_types=[
                pltpu.VMEM((_CH,), jnp.int32),
                pltpu.VMEM((_CH,), jnp.int32),
                pltpu.VMEM((_CH, lat), jnp.float32),
                pltpu.VMEM((_CH, lat), jnp.float32),
                pltpu.SemaphoreType.DMA,
                pltpu.SemaphoreType.DMA,
            ],
        )

    idx_parts = []
    zq_parts = []
    loss_parts = []
    base = 0
    for nbh in _SPLIT_BLOCKS:
        nh = nbh * _BT
        idx3, loss_sum = pl.pallas_call(
            _tc_body,
            grid=(nbh,),
            in_specs=[
                pl.BlockSpec((_BT, in_dim), lambda i, base=base: (base + i, 0)),
                pl.BlockSpec(W1.shape, lambda i: (0, 0)),
                pl.BlockSpec(b1.shape, lambda i: (0,)),
                pl.BlockSpec(g1.shape, lambda i: (0,)),
                pl.BlockSpec(be1.shape, lambda i: (0,)),
                pl.BlockSpec(W2.shape, lambda i: (0, 0)),
                pl.BlockSpec(b2.shape, lambda i: (0,)),
                pl.BlockSpec(g2.shape, lambda i: (0,)),
                pl.BlockSpec(be2.shape, lambda i: (0,)),
                pl.BlockSpec(codebook.shape, lambda i: (0, 0)),
            ],
            out_specs=[
                pl.BlockSpec((1, 1, _BT), lambda i: (i, 0, 0)),
                pl.BlockSpec((1, 1), lambda i: (0, 0)),
            ],
            out_shape=[
                jax.ShapeDtypeStruct((nbh, 1, _BT), jnp.int32),
                jax.ShapeDtypeStruct((1, 1), jnp.float32),
            ],
        )(x, W1, b1, g1, be1, W2, b2, g2, be2, codebook)

        idx_flat = idx3.reshape(nh)
        idx_parts.append(idx_flat)
        loss_parts.append(loss_sum[0, 0])
        zq_parts.append(make_sc_gather(nh)(codebook, idx_flat))
        base += nbh

    zq = jnp.concatenate(zq_parts, axis=0)
    idx_all = jnp.concatenate(idx_parts, axis=0)
    loss = sum(loss_parts) * (1.25 / (n * lat))
    return zq.reshape(b, t, lat), loss, idx_all.reshape(b, t)


# speedup vs baseline: 1.0492x; 1.0492x over previous
import functools
import math

import jax
import jax.numpy as jnp
from jax import lax
from jax.experimental import pallas as pl
from jax.experimental.pallas import tpu as pltpu
from jax.experimental.pallas import tpu_sc as plsc


_BT = 1024
_NC, _NS = 2, 16
_NW = _NC * _NS
_CH = 256


def _tc_body(x_ref, w1_ref, b1_ref, g1_ref, be1_ref, w2_ref, b2_ref, g2_ref,
             be2_ref, cb_ref, idx_ref, loss_ref):
    x = x_ref[...]
    h = jnp.dot(x, w1_ref[...], preferred_element_type=jnp.float32)
    h = h + b1_ref[...]
    mu = jnp.mean(h, axis=-1, keepdims=True)
    var = jnp.mean((h - mu) ** 2, axis=-1, keepdims=True)
    h = (h - mu) * jax.lax.rsqrt(var + 1e-5) * g1_ref[...] + be1_ref[...]
    h = 0.5 * h * (1.0 + jax.lax.erf(h * (1.0 / math.sqrt(2.0))))
    z = jnp.dot(h, w2_ref[...], preferred_element_type=jnp.float32)
    z = z + b2_ref[...]
    mu2 = jnp.mean(z, axis=-1, keepdims=True)
    var2 = jnp.mean((z - mu2) ** 2, axis=-1, keepdims=True)
    z = (z - mu2) * jax.lax.rsqrt(var2 + 1e-5) * g2_ref[...] + be2_ref[...]

    cb = cb_ref[...]
    cb2 = jnp.sum(cb * cb, axis=1)
    z2 = jnp.sum(z * z, axis=1, keepdims=True)
    dist = (z2 + cb2[None, :]) - jnp.dot(z + z, cb.T, preferred_element_type=jnp.float32)
    k = cb.shape[0]
    iota = jax.lax.broadcasted_iota(jnp.int32, (1, k), 1).astype(jnp.float32)
    minv = jnp.min(dist, axis=1, keepdims=True)
    idx = jnp.min(jnp.where(dist == minv, iota, float(k)), axis=1).astype(jnp.int32)
    idx_ref[0, 0, :] = idx

    part = jnp.sum(minv).reshape(1, 1)

    @pl.when(pl.program_id(0) == 0)
    def _init():
        loss_ref[...] = jnp.zeros_like(loss_ref)

    loss_ref[...] += part


def _sc_gather(cb_hbm, idx_hbm, zq_hbm, idx0, idx1, rows0, rows1, sem0, sem1):
    n = zq_hbm.shape[0]
    bpw = n // _NW
    nch = bpw // _CH
    wid = lax.axis_index("s") * _NC + lax.axis_index("c")
    base = wid * bpw
    idx_v = (idx0, idx1)
    rows_v = (rows0, rows1)
    sems = (sem0, sem1)

    pltpu.sync_copy(idx_hbm.at[pl.ds(base, _CH)], idx0)
    gathers = [pltpu.async_copy(cb_hbm.at[idx0], rows0, sem0)]
    for c in range(nch):
        cur = c % 2
        nxt = (c + 1) % 2
        if c + 1 < nch:
            off = base + (c + 1) * _CH
            pltpu.sync_copy(idx_hbm.at[pl.ds(off, _CH)], idx_v[nxt])
            gathers.append(
                pltpu.async_copy(cb_hbm.at[idx_v[nxt]], rows_v[nxt], sems[nxt]))
        gathers[c].wait()
        pltpu.sync_copy(rows_v[cur], zq_hbm.at[pl.ds(base + c * _CH, _CH)])


_SPLIT_BLOCKS = (32, 32)


def kernel(embeddings, W1, b1, g1, be1, W2, b2, g2, be2, codebook):
    b, t, in_dim = embeddings.shape
    n = b * t
    lat = codebook.shape[1]
    x = embeddings.reshape(n, in_dim)

    def make_sc_gather(nh):
        return pl.kernel(
            _sc_gather,
            mesh=plsc.VectorSubcoreMesh(core_axis_name="c", subcore_axis_name="s"),
            compiler_params=pltpu.CompilerParams(use_tc_tiling_on_sc=False),
            out_type=jax.ShapeDtypeStruct((nh, lat), jnp.float32),
            scratch_types=[
                pltpu.VMEM((_CH,), jnp.int32),
                pltpu.VMEM((_CH,), jnp.int32),
                pltpu.VMEM((_CH, lat), jnp.float32),
                pltpu.VMEM((_CH, lat), jnp.float32),
                pltpu.SemaphoreType.DMA,
                pltpu.SemaphoreType.DMA,
            ],
        )

    idx_parts = []
    zq_parts = []
    loss_parts = []
    base = 0
    for nbh in _SPLIT_BLOCKS:
        nh = nbh * _BT
        idx3, loss_sum = pl.pallas_call(
            _tc_body,
            grid=(nbh,),
            in_specs=[
                pl.BlockSpec((_BT, in_dim), lambda i, base=base: (base + i, 0)),
                pl.BlockSpec(W1.shape, lambda i: (0, 0)),
                pl.BlockSpec(b1.shape, lambda i: (0,)),
                pl.BlockSpec(g1.shape, lambda i: (0,)),
                pl.BlockSpec(be1.shape, lambda i: (0,)),
                pl.BlockSpec(W2.shape, lambda i: (0, 0)),
                pl.BlockSpec(b2.shape, lambda i: (0,)),
                pl.BlockSpec(g2.shape, lambda i: (0,)),
                pl.BlockSpec(be2.shape, lambda i: (0,)),
                pl.BlockSpec(codebook.shape, lambda i: (0, 0)),
            ],
            out_specs=[
                pl.BlockSpec((1, 1, _BT), lambda i: (i, 0, 0)),
                pl.BlockSpec((1, 1), lambda i: (0, 0)),
            ],
            out_shape=[
                jax.ShapeDtypeStruct((nbh, 1, _BT), jnp.int32),
                jax.ShapeDtypeStruct((1, 1), jnp.float32),
            ],
        )(x, W1, b1, g1, be1, W2, b2, g2, be2, codebook)

        idx_flat = idx3.reshape(nh)
        idx_parts.append(idx_flat)
        loss_parts.append(loss_sum[0, 0])
        zq_parts.append(make_sc_gather(nh)(codebook, idx_flat))
        base += nbh

    zq = jnp.concatenate(zq_parts, axis=0)
    idx_all = jnp.concatenate(idx_parts, axis=0)
    loss = sum(loss_parts) * (1.25 / (n * lat))
    return zq.reshape(b, t, lat), loss, idx_all.reshape(b, t)
